# baseline (speedup 1.0000x reference)
import jax
import jax.numpy as jnp
from jax import lax
from jax.experimental import pallas as pl
from jax.experimental.pallas import tpu as pltpu
from jax.experimental.pallas import tpu_sc as plsc

B = 4
N = 4096
M = 1024
C = 64
NSAMPLE = 32
R2 = 0.3 * 0.3
L = 16
NCHUNK = N // L
NTILES = 32
M_PER_TILE = (B * M) // NTILES
TILES_PER_BATCH = M // M_PER_TILE
G = 4
NGROUP = M_PER_TILE // G
GR = G * NSAMPLE
CP = 2 * C
U = 4


def _sc_body(xyz_hbm, newxyz_hbm, feat_hbm, outxyz_hbm, outfeat_hbm,
             xyz_v, newxyz_v, selbuf, idxbuf, xyzout_v,
             rows0, rows1, rows2, rows3,
             gsem0, gsem1, gsem2, gsem3, osem0, osem1, osem2, osem3, xsem):
    wid = lax.axis_index("s") * 2 + lax.axis_index("c")
    b = wid // TILES_PER_BATCH
    m0 = (wid % TILES_PER_BATCH) * M_PER_TILE

    pltpu.sync_copy(xyz_hbm.at[b], xyz_v)
    for coord in range(3):
        pltpu.sync_copy(
            newxyz_hbm.at[b, pl.ds(coord * M + m0, M_PER_TILE)],
            newxyz_v.at[pl.ds(coord * M_PER_TILE, M_PER_TILE)])

    iota = lax.iota(jnp.int32, L)
    zeros16 = jnp.zeros((L,), jnp.int32)
    r2 = jnp.float32(R2)

    def per_centroid(ml, _):
        mlv = jnp.full((L,), ml, jnp.int32)
        qx = plsc.load_gather(newxyz_v, [mlv])
        qy = plsc.load_gather(newxyz_v, [mlv + M_PER_TILE])
        qz = plsc.load_gather(newxyz_v, [mlv + 2 * M_PER_TILE])

        def cond(carry):
            j, cntv = carry
            return jnp.logical_and(j < NCHUNK, jnp.max(cntv) < NSAMPLE)

        def step(carry):
            j, cntv = carry
            off0 = pl.multiple_of(j * L, U * L)
            for u in range(U):
                off = off0 + u * L
                px = xyz_v[pl.ds(off, L)]
                py = xyz_v[pl.ds(off + N, L)]
                pz = xyz_v[pl.ds(off + 2 * N, L)]
                dx = px - qx
                dy = py - qy
                dz = pz - qz
                d2 = dx * dx + dy * dy + dz * dz
                msk = d2 < r2
                incl = plsc.cumsum(msk.astype(jnp.int32))
                pos = cntv + incl - 1
                idxv = (j + u) * L + iota
                plsc.store_scatter(selbuf, [pos], idxv, mask=msk)
                cntv = cntv + plsc.all_reduce_population_count(msk)
            return j + U, cntv

        _, cntv = lax.while_loop(
            cond, step, (jnp.int32(0), jnp.zeros((L,), jnp.int32)))
        cnt = jnp.max(cntv)

        s0 = selbuf[pl.ds(0, L)]
        s1 = selbuf[pl.ds(L, L)]
        cv = jnp.full((L,), cnt, jnp.int32)
        first = plsc.load_gather(selbuf, [zeros16])
        first = jnp.where(cv > 0, first, jnp.full((L,), N - 1, jnp.int32))
        sel0 = jnp.where(iota < cv, s0, first)
        sel1 = jnp.where(iota + L < cv, s1, first)

        o = ml * (3 * NSAMPLE)
        pos0 = iota * 3 + o
        pos1 = pos0 + 3 * L
        for coord, q in ((0, qx), (1, qy), (2, qz)):
            g0 = plsc.load_gather(xyz_v, [sel0 + coord * N])
            g1 = plsc.load_gather(xyz_v, [sel1 + coord * N])
            plsc.store_scatter(xyzout_v, [pos0 + coord], g0 - q)
            plsc.store_scatter(xyzout_v, [pos1 + coord], g1 - q)

        idxbuf[pl.ds(ml * NSAMPLE, L)] = sel0 + b * N
        idxbuf[pl.ds(ml * NSAMPLE + L, L)] = sel1 + b * N
        return _

    lax.fori_loop(0, M_PER_TILE, per_centroid, 0)

    xyzdma = pltpu.async_copy(
        xyzout_v,
        outxyz_hbm.at[b, pl.ds(m0 * (3 * NSAMPLE), M_PER_TILE * 3 * NSAMPLE)],
        xsem)

    rows = (rows0, rows1, rows2, rows3)
    gsems = (gsem0, gsem1, gsem2, gsem3)
    osems = (osem0, osem1, osem2, osem3)
    NS4 = NGROUP // 4

    def gather(g, s):
        return pltpu.async_copy(
            feat_hbm.at[idxbuf.at[pl.ds(g * GR, GR)]], rows[s], gsems[s])

    def wait_gather(s):
        pltpu.make_async_copy(
            feat_hbm.at[idxbuf.at[pl.ds(0, GR)]], rows[s], gsems[s]).wait()

    def flush(g, s):
        return pltpu.async_copy(
            rows[s], outfeat_hbm.at[b, pl.ds((m0 + g * G) * NSAMPLE, GR), :],
            osems[s])

    def drain_flush(s):
        pltpu.make_async_copy(
            rows[s], outfeat_hbm.at[b, pl.ds(m0 * NSAMPLE, GR), :],
            osems[s]).wait()

    def ring(kk, _):
        for s in range(4):
            @pl.when(kk > 0)
            def _d():
                drain_flush(s)
            gather(4 * kk + s, s)
        for s in range(4):
            wait_gather(s)
            flush(4 * kk + s, s)
        return _

    lax.fori_loop(0, NS4, ring, 0)
    for s in range(4):
        drain_flush(s)
    xyzdma.wait()


@jax.jit
def _run(xyz_t, newxyz_t, feat_rows):
    mesh = plsc.VectorSubcoreMesh(core_axis_name="c", subcore_axis_name="s")
    f = pl.kernel(
        _sc_body,
        out_type=(
            jax.ShapeDtypeStruct((B, M * NSAMPLE * 3), jnp.float32),
            jax.ShapeDtypeStruct((B, M * NSAMPLE, CP), jnp.float32),
        ),
        mesh=mesh,
        compiler_params=pltpu.CompilerParams(needs_layout_passes=False),
        scratch_types=[
            pltpu.VMEM((3 * N,), jnp.float32),
            pltpu.VMEM((3 * M_PER_TILE,), jnp.float32),
            pltpu.VMEM((NSAMPLE + U * L,), jnp.int32),
            pltpu.VMEM((M_PER_TILE * NSAMPLE,), jnp.int32),
            pltpu.VMEM((M_PER_TILE * 3 * NSAMPLE,), jnp.float32),
            pltpu.VMEM((GR, CP), jnp.float32),
            pltpu.VMEM((GR, CP), jnp.float32),
            pltpu.VMEM((GR, CP), jnp.float32),
            pltpu.VMEM((GR, CP), jnp.float32),
            pltpu.SemaphoreType.DMA,
            pltpu.SemaphoreType.DMA,
            pltpu.SemaphoreType.DMA,
            pltpu.SemaphoreType.DMA,
            pltpu.SemaphoreType.DMA,
            pltpu.SemaphoreType.DMA,
            pltpu.SemaphoreType.DMA,
            pltpu.SemaphoreType.DMA,
            pltpu.SemaphoreType.DMA,
        ],
    )
    return f(xyz_t, newxyz_t, feat_rows)


def kernel(xyz, new_xyz, features):
    xyz_t = jnp.transpose(xyz, (0, 2, 1)).reshape(B, 3 * N)
    newxyz_t = jnp.transpose(new_xyz, (0, 2, 1)).reshape(B, 3 * M)
    feat_rows = jnp.pad(jnp.transpose(features, (0, 2, 1)).reshape(B * N, C),
                        ((0, 0), (0, CP - C)))
    out_xyz, out_feat = _run(xyz_t, newxyz_t, feat_rows)
    xyz4 = out_xyz.reshape(B, M, NSAMPLE, 3)
    feat4 = out_feat.reshape(B, M, NSAMPLE, CP)[..., :C]
    cat = jnp.concatenate([xyz4, feat4], axis=-1)
    return jnp.transpose(cat, (0, 3, 1, 2))


# speedup vs baseline: 1.2664x; 1.2664x over previous
import jax
import jax.numpy as jnp
from jax import lax
from jax.experimental import pallas as pl
from jax.experimental.pallas import tpu as pltpu
from jax.experimental.pallas import tpu_sc as plsc

B = 4
N = 4096
M = 1024
C = 64
NSAMPLE = 32
R2 = 0.3 * 0.3
L = 16
NCHUNK = N // L
NTILES = 32
M_PER_TILE = (B * M) // NTILES
TILES_PER_BATCH = M // M_PER_TILE
G = 4
NGROUP = M_PER_TILE // G
GR = G * NSAMPLE
U = 4


def _sc_body(xyz_hbm, newxyz_hbm, feat_hbm, outxyz_hbm, outfeat_hbm,
             xyz_v, newxyz_v, selbuf, idxbuf, xyzout_v, rowsbuf,
             gsem0, gsem1, osem0, osem1, xsem):
    wid = lax.axis_index("s") * 2 + lax.axis_index("c")
    b = wid // TILES_PER_BATCH
    m0 = (wid % TILES_PER_BATCH) * M_PER_TILE

    pltpu.sync_copy(xyz_hbm.at[b], xyz_v)
    for coord in range(3):
        pltpu.sync_copy(
            newxyz_hbm.at[b, pl.ds(coord * M + m0, M_PER_TILE)],
            newxyz_v.at[pl.ds(coord * M_PER_TILE, M_PER_TILE)])

    iota = lax.iota(jnp.int32, L)
    zeros16 = jnp.zeros((L,), jnp.int32)
    r2 = jnp.float32(R2)

    def per_centroid(ml, _):
        mlv = jnp.full((L,), ml, jnp.int32)
        qx = plsc.load_gather(newxyz_v, [mlv])
        qy = plsc.load_gather(newxyz_v, [mlv + M_PER_TILE])
        qz = plsc.load_gather(newxyz_v, [mlv + 2 * M_PER_TILE])

        def cond(carry):
            j, cntv = carry
            return jnp.logical_and(j < NCHUNK, jnp.max(cntv) < NSAMPLE)

        def step(carry):
            j, cntv = carry
            off0 = pl.multiple_of(j * L, U * L)
            for u in range(U):
                off = off0 + u * L
                px = xyz_v[pl.ds(off, L)]
                py = xyz_v[pl.ds(off + N, L)]
                pz = xyz_v[pl.ds(off + 2 * N, L)]
                dx = px - qx
                dy = py - qy
                dz = pz - qz
                d2 = dx * dx + dy * dy + dz * dz
                msk = d2 < r2
                incl = plsc.cumsum(msk.astype(jnp.int32))
                pos = cntv + incl - 1
                idxv = (j + u) * L + iota
                plsc.store_scatter(selbuf, [pos], idxv, mask=msk)
                cntv = cntv + plsc.all_reduce_population_count(msk)
            return j + U, cntv

        _, cntv = lax.while_loop(
            cond, step, (jnp.int32(0), jnp.zeros((L,), jnp.int32)))
        cnt = jnp.max(cntv)

        s0 = selbuf[pl.ds(0, L)]
        s1 = selbuf[pl.ds(L, L)]
        cv = jnp.full((L,), cnt, jnp.int32)
        first = plsc.load_gather(selbuf, [zeros16])
        first = jnp.where(cv > 0, first, jnp.full((L,), N - 1, jnp.int32))
        sel0 = jnp.where(iota < cv, s0, first)
        sel1 = jnp.where(iota + L < cv, s1, first)

        o = ml * NSAMPLE
        for coord, q in ((0, qx), (1, qy), (2, qz)):
            g0 = plsc.load_gather(xyz_v, [sel0 + coord * N])
            g1 = plsc.load_gather(xyz_v, [sel1 + coord * N])
            xyzout_v[coord, pl.ds(o, L)] = g0 - q
            xyzout_v[coord, pl.ds(o + L, L)] = g1 - q

        idxbuf[pl.ds(ml * NSAMPLE, L)] = sel0 + b * N
        idxbuf[pl.ds(ml * NSAMPLE + L, L)] = sel1 + b * N
        return _

    lax.fori_loop(0, M_PER_TILE, per_centroid, 0)

    xyzdma = pltpu.async_copy(
        xyzout_v, outxyz_hbm.at[b, :, pl.ds(m0 * NSAMPLE, M_PER_TILE * NSAMPLE)],
        xsem)

    rows = (rowsbuf.at[0], rowsbuf.at[1])
    gsems = (gsem0, gsem1)
    osems = (osem0, osem1)

    def gather(g, p):
        return pltpu.async_copy(
            feat_hbm.at[idxbuf.at[pl.ds(g * GR, GR)]], rows[p], gsems[p])

    def flush(g, p):
        return pltpu.async_copy(
            rows[p],
            outfeat_hbm.at[b, pl.ds((m0 + g * G) * NSAMPLE, GR)],
            osems[p])

    gdma = [gather(0, 0), None]
    fdma = [None, None]
    for g in range(NGROUP):
        p = g & 1
        q = p ^ 1
        if g + 1 < NGROUP:
            if fdma[q] is not None:
                fdma[q].wait()
            gdma[q] = gather(g + 1, q)
        gdma[p].wait()
        fdma[p] = flush(g, p)
    fdma[0].wait()
    fdma[1].wait()
    xyzdma.wait()


@jax.jit
def _run(xyz_t, newxyz_t, feat_rows):
    mesh = plsc.VectorSubcoreMesh(core_axis_name="c", subcore_axis_name="s")
    f = pl.kernel(
        _sc_body,
        out_type=(
            jax.ShapeDtypeStruct((B, 3, M * NSAMPLE), jnp.float32),
            jax.ShapeDtypeStruct((B, M * NSAMPLE, C), jnp.float32),
        ),
        mesh=mesh,
        compiler_params=pltpu.CompilerParams(
            needs_layout_passes=False, use_tc_tiling_on_sc=False),
        scratch_types=[
            pltpu.VMEM((3 * N,), jnp.float32),
            pltpu.VMEM((3 * M_PER_TILE,), jnp.float32),
            pltpu.VMEM((NSAMPLE + U * L,), jnp.int32),
            pltpu.VMEM((M_PER_TILE * NSAMPLE,), jnp.int32),
            pltpu.VMEM((3, M_PER_TILE * NSAMPLE), jnp.float32),
            pltpu.VMEM((2, GR, C), jnp.float32),
            pltpu.SemaphoreType.DMA,
            pltpu.SemaphoreType.DMA,
            pltpu.SemaphoreType.DMA,
            pltpu.SemaphoreType.DMA,
            pltpu.SemaphoreType.DMA,
        ],
    )
    return f(xyz_t, newxyz_t, feat_rows)


def kernel(xyz, new_xyz, features):
    xyz_t = jnp.transpose(xyz, (0, 2, 1)).reshape(B, 3 * N)
    newxyz_t = jnp.transpose(new_xyz, (0, 2, 1)).reshape(B, 3 * M)
    feat_rows = jnp.transpose(features, (0, 2, 1)).reshape(B * N, C)
    out_xyz, out_feat = _run(xyz_t, newxyz_t, feat_rows)
    out_xyz = out_xyz.reshape(B, 3, M, NSAMPLE)
    grouped_feat = jnp.transpose(
        out_feat.reshape(B, M, NSAMPLE, C), (0, 3, 1, 2))
    return jnp.concatenate([out_xyz, grouped_feat], axis=1)


# speedup vs baseline: 1.3782x; 1.0883x over previous
import jax
import jax.numpy as jnp
from jax import lax
from jax.experimental import pallas as pl
from jax.experimental.pallas import tpu as pltpu
from jax.experimental.pallas import tpu_sc as plsc

B = 4
N = 4096
M = 1024
C = 64
NSAMPLE = 32
R2 = 0.3 * 0.3
L = 16
NCHUNK = N // L
NTILES = 32
M_PER_TILE = (B * M) // NTILES
TILES_PER_BATCH = M // M_PER_TILE
G = 4
NGROUP = M_PER_TILE // G
GR = G * NSAMPLE


def _sc_body(xyz_hbm, newxyz_hbm, feat_hbm, outxyz_hbm, outfeat_hbm,
             xyz_v, newxyz_v, selbuf, idxbuf, xyzout_v, rowsbuf,
             gsem0, gsem1, osem0, osem1, xsem):
    wid = lax.axis_index("s") * 2 + lax.axis_index("c")
    b = wid // TILES_PER_BATCH
    m0 = (wid % TILES_PER_BATCH) * M_PER_TILE

    pltpu.sync_copy(xyz_hbm.at[b], xyz_v)
    for coord in range(3):
        pltpu.sync_copy(
            newxyz_hbm.at[b, pl.ds(coord * M + m0, M_PER_TILE)],
            newxyz_v.at[pl.ds(coord * M_PER_TILE, M_PER_TILE)])

    iota = lax.iota(jnp.int32, L)
    zeros16 = jnp.zeros((L,), jnp.int32)
    r2 = jnp.float32(R2)

    def per_centroid(ml, _):
        mlv = jnp.full((L,), ml, jnp.int32)
        qx = plsc.load_gather(newxyz_v, [mlv])
        qy = plsc.load_gather(newxyz_v, [mlv + M_PER_TILE])
        qz = plsc.load_gather(newxyz_v, [mlv + 2 * M_PER_TILE])

        def cond(carry):
            j, cnt = carry
            return jnp.logical_and(j < NCHUNK, cnt < NSAMPLE)

        def step(carry):
            j, cnt = carry
            off = pl.multiple_of(j * L, 2 * L)
            pa = [xyz_v[pl.ds(off + coord * N, L)] for coord in range(3)]
            pb = [xyz_v[pl.ds(off + coord * N + L, L)] for coord in range(3)]
            da = [pa[0] - qx, pa[1] - qy, pa[2] - qz]
            db = [pb[0] - qx, pb[1] - qy, pb[2] - qz]
            d2a = da[0] * da[0] + da[1] * da[1] + da[2] * da[2]
            d2b = db[0] * db[0] + db[1] * db[1] + db[2] * db[2]
            mska = d2a < r2
            mskb = d2b < r2
            popa = jnp.max(plsc.all_reduce_population_count(mska))
            popb = jnp.max(plsc.all_reduce_population_count(mskb))
            idxv = j * L + iota
            plsc.store_compressed(selbuf.at[pl.ds(cnt, L)], idxv, mask=mska)
            plsc.store_compressed(selbuf.at[pl.ds(cnt + popa, L)],
                                  idxv + L, mask=mskb)
            return j + 2, cnt + popa + popb

        _, cnt = lax.while_loop(cond, step, (jnp.int32(0), jnp.int32(0)))

        s0 = selbuf[pl.ds(0, L)]
        s1 = selbuf[pl.ds(L, L)]
        cv = jnp.full((L,), cnt, jnp.int32)
        first = plsc.load_gather(selbuf, [zeros16])
        first = jnp.where(cv > 0, first, jnp.full((L,), N - 1, jnp.int32))
        sel0 = jnp.where(iota < cv, s0, first)
        sel1 = jnp.where(iota + L < cv, s1, first)

        o = ml * NSAMPLE
        for coord, q in ((0, qx), (1, qy), (2, qz)):
            g0 = plsc.load_gather(xyz_v, [sel0 + coord * N])
            g1 = plsc.load_gather(xyz_v, [sel1 + coord * N])
            xyzout_v[coord, pl.ds(o, L)] = g0 - q
            xyzout_v[coord, pl.ds(o + L, L)] = g1 - q

        idxbuf[pl.ds(ml * NSAMPLE, L)] = sel0 + b * N
        idxbuf[pl.ds(ml * NSAMPLE + L, L)] = sel1 + b * N
        return _

    lax.fori_loop(0, M_PER_TILE, per_centroid, 0)

    xyzdma = pltpu.async_copy(
        xyzout_v, outxyz_hbm.at[b, :, pl.ds(m0 * NSAMPLE, M_PER_TILE * NSAMPLE)],
        xsem)

    rows = (rowsbuf.at[0], rowsbuf.at[1])
    gsems = (gsem0, gsem1)
    osems = (osem0, osem1)

    def gather(g, p):
        return pltpu.async_copy(
            feat_hbm.at[idxbuf.at[pl.ds(g * GR, GR)]], rows[p], gsems[p])

    def flush(g, p):
        return pltpu.async_copy(
            rows[p],
            outfeat_hbm.at[b, pl.ds((m0 + g * G) * NSAMPLE, GR)],
            osems[p])

    gdma = [gather(0, 0), None]
    fdma = [None, None]
    for g in range(NGROUP):
        p = g & 1
        q = p ^ 1
        if g + 1 < NGROUP:
            if fdma[q] is not None:
                fdma[q].wait()
            gdma[q] = gather(g + 1, q)
        gdma[p].wait()
        fdma[p] = flush(g, p)
    fdma[0].wait()
    fdma[1].wait()
    xyzdma.wait()


@jax.jit
def _run(xyz_t, newxyz_t, feat_rows):
    mesh = plsc.VectorSubcoreMesh(core_axis_name="c", subcore_axis_name="s")
    f = pl.kernel(
        _sc_body,
        out_type=(
            jax.ShapeDtypeStruct((B, 3, M * NSAMPLE), jnp.float32),
            jax.ShapeDtypeStruct((B, M * NSAMPLE, C), jnp.float32),
        ),
        mesh=mesh,
        compiler_params=pltpu.CompilerParams(
            needs_layout_passes=False, use_tc_tiling_on_sc=False),
        scratch_types=[
            pltpu.VMEM((3 * N,), jnp.float32),
            pltpu.VMEM((3 * M_PER_TILE,), jnp.float32),
            pltpu.VMEM((80,), jnp.int32),
            pltpu.VMEM((M_PER_TILE * NSAMPLE,), jnp.int32),
            pltpu.VMEM((3, M_PER_TILE * NSAMPLE), jnp.float32),
            pltpu.VMEM((2, GR, C), jnp.float32),
            pltpu.SemaphoreType.DMA,
            pltpu.SemaphoreType.DMA,
            pltpu.SemaphoreType.DMA,
            pltpu.SemaphoreType.DMA,
            pltpu.SemaphoreType.DMA,
        ],
    )
    return f(xyz_t, newxyz_t, feat_rows)


def kernel(xyz, new_xyz, features):
    xyz_t = jnp.transpose(xyz, (0, 2, 1)).reshape(B, 3 * N)
    newxyz_t = jnp.transpose(new_xyz, (0, 2, 1)).reshape(B, 3 * M)
    feat_rows = jnp.transpose(features, (0, 2, 1)).reshape(B * N, C)
    out_xyz, out_feat = _run(xyz_t, newxyz_t, feat_rows)
    out_xyz = out_xyz.reshape(B, 3, M, NSAMPLE)
    grouped_feat = jnp.transpose(
        out_feat.reshape(B, M, NSAMPLE, C), (0, 3, 1, 2))
    return jnp.concatenate([out_xyz, grouped_feat], axis=1)
